# baseline (speedup 1.0000x reference)
import functools

import jax
import jax.numpy as jnp
from jax import lax
from jax.experimental import pallas as pl
from jax.experimental.pallas import tpu as pltpu
from jax.experimental.pallas import tpu_sc as plsc

B_SEG = 2048
NC, NS = 2, 16
NW = NC * NS
SEG_PER_W = B_SEG // NW
CHUNK = 512
LANES = 16
NVEC = 8
NEG_INF = float("-inf")


def _sc_segment_reduce(x, m32, edges):
    N, D = x.shape
    mesh = plsc.VectorSubcoreMesh(core_axis_name="c", subcore_axis_name="s")

    @functools.partial(
        pl.kernel,
        out_type=(
            jax.ShapeDtypeStruct((B_SEG, D), jnp.float32),
            jax.ShapeDtypeStruct((B_SEG, D), jnp.float32),
        ),
        mesh=mesh,
        scratch_types=[
            pltpu.VMEM((CHUNK, D), jnp.float32),
            pltpu.VMEM((CHUNK + LANES,), jnp.int32),
            pltpu.VMEM((SEG_PER_W, D), jnp.float32),
            pltpu.VMEM((SEG_PER_W, D), jnp.float32),
            pltpu.VMEM((SEG_PER_W + 2 * LANES,), jnp.int32),
        ],
    )
    def seg_kernel(x_hbm, m_hbm, edges_hbm, sum_hbm, max_hbm,
                   xbuf, mbuf, acc_s, acc_m, e_v):
        w = lax.axis_index("s") * NC + lax.axis_index("c")
        seg_lo = w * SEG_PER_W
        pltpu.sync_copy(edges_hbm.at[pl.ds(seg_lo, SEG_PER_W + 2 * LANES)], e_v)
        r0 = e_v[pl.ds(0, LANES)][0]
        r1 = e_v[pl.ds(SEG_PER_W, LANES)][0]

        zeros = jnp.zeros((LANES,), jnp.float32)
        ninf = jnp.full((LANES,), NEG_INF, jnp.float32)

        def init_body(i, _):
            s = i // NVEC
            j = i % NVEC
            acc_s[s, pl.ds(j * LANES, LANES)] = zeros
            acc_m[s, pl.ds(j * LANES, LANES)] = ninf
            return 0

        lax.fori_loop(0, SEG_PER_W * NVEC, init_body, 0)

        a0 = (r0 // 8) * 8
        nchunks = (r1 - a0 + CHUNK - 1) // CHUNK

        @pl.loop(0, nchunks, init_carry=jnp.int32(0))
        def _chunks(k, cur):
            s_un = a0 + k * CHUNK
            c0 = jnp.minimum(s_un, N - CHUNK)
            pltpu.sync_copy(x_hbm.at[pl.ds(c0, CHUNK)], xbuf)
            pltpu.sync_copy(m_hbm.at[pl.ds(c0, CHUNK)], mbuf.at[pl.ds(0, CHUNK)])
            lo = jnp.maximum(r0, s_un) - c0
            hi = jnp.minimum(r1, s_un + CHUNK) - c0

            hi0 = jnp.maximum(hi - 1, 0)
            m_last = mbuf[pl.ds(hi0, LANES)][0] - seg_lo
            m_last = jnp.minimum(jnp.maximum(m_last, cur - 1),
                                 SEG_PER_W - 1)

            @pl.loop(cur, m_last + 1)
            def _runs(si):
                st = jnp.maximum(e_v[pl.ds(si, LANES)][0] - c0, lo)
                en = jnp.minimum(e_v[pl.ds(si + 1, LANES)][0] - c0, hi)

                accs = tuple(acc_s[si, pl.ds(j * LANES, LANES)]
                             for j in range(NVEC))
                accm = tuple(acc_m[si, pl.ds(j * LANES, LANES)]
                             for j in range(NVEC))

                @plsc.parallel_loop(st, en, unroll=8, carry=accs + accm)
                def out(r, carry):
                    a = carry[:NVEC]
                    m = carry[NVEC:]
                    vs = tuple(xbuf[r, pl.ds(j * LANES, LANES)]
                               for j in range(NVEC))
                    a = tuple(a[j] + vs[j] for j in range(NVEC))
                    m = tuple(jnp.maximum(m[j], vs[j]) for j in range(NVEC))
                    return a + m

                for j in range(NVEC):
                    acc_s[si, pl.ds(j * LANES, LANES)] = out[j]
                    acc_m[si, pl.ds(j * LANES, LANES)] = out[NVEC + j]

            return jnp.maximum(m_last, cur)

        pltpu.sync_copy(acc_s, sum_hbm.at[pl.ds(seg_lo, SEG_PER_W)])
        pltpu.sync_copy(acc_m, max_hbm.at[pl.ds(seg_lo, SEG_PER_W)])

    return seg_kernel(x, m32, edges)


def _tc_merge(seg_max, seg_sum, W_merge, b_merge):
    B, D = seg_max.shape

    def body(mx_ref, sm_ref, w_ref, b_ref, o_ref):
        mx = mx_ref[...]
        mx = jnp.where(jnp.isfinite(mx), mx, 0.0)
        acc = jnp.dot(mx, w_ref[0:D, :], preferred_element_type=jnp.float32)
        acc = acc + jnp.dot(sm_ref[...], w_ref[D:2 * D, :],
                            preferred_element_type=jnp.float32)
        o_ref[...] = acc + b_ref[...]

    return pl.pallas_call(
        body,
        out_shape=jax.ShapeDtypeStruct((B, W_merge.shape[1]), jnp.float32),
    )(seg_max, seg_sum, W_merge, b_merge)


def kernel(x, membership, W_merge, b_merge):
    m32 = membership.astype(jnp.int32)
    edges = jnp.searchsorted(
        m32, jnp.arange(B_SEG + 1, dtype=jnp.int32), side="left"
    ).astype(jnp.int32)
    edges = jnp.pad(edges, (0, 2 * LANES - 1), constant_values=2 ** 30)
    seg_sum, seg_max = _sc_segment_reduce(x, m32, edges)
    return _tc_merge(seg_max, seg_sum, W_merge, jnp.reshape(b_merge, (1, -1)))


# speedup vs baseline: 6.6616x; 6.6616x over previous
import functools

import jax
import jax.numpy as jnp
from jax import lax
from jax.experimental import pallas as pl
from jax.experimental.pallas import tpu as pltpu
from jax.experimental.pallas import tpu_sc as plsc

B_SEG = 2048
NC, NS = 2, 16
NW = NC * NS
SEG_PER_W = B_SEG // NW
CHUNK = 512
LANES = 16
NVEC = 8
TBL = 2304
SLICE = 96
NEG_INF = float("-inf")
BIG = jnp.int32(2 ** 30)


def _sc_graph_readout(x, m32):
    N, D = x.shape
    cnt = N // NS
    mesh = plsc.VectorSubcoreMesh(core_axis_name="c", subcore_axis_name="s")

    @functools.partial(
        pl.kernel,
        out_type=(
            jax.ShapeDtypeStruct((B_SEG, D), jnp.float32),
            jax.ShapeDtypeStruct((B_SEG, D), jnp.float32),
        ),
        mesh=mesh,
        compiler_params=pltpu.CompilerParams(needs_layout_passes=False),
        scratch_types=[
            pltpu.VMEM((CHUNK, D), jnp.float32),
            pltpu.VMEM((CHUNK + LANES,), jnp.int32),
            pltpu.VMEM((SEG_PER_W, D), jnp.float32),
            pltpu.VMEM((SEG_PER_W, D), jnp.float32),
            pltpu.VMEM((cnt + 2 * LANES,), jnp.int32),
            pltpu.VMEM((TBL,), jnp.int32),
            pltpu.VMEM((TBL,), jnp.int32),
            pltpu.VMEM_SHARED((NS * TBL,), jnp.int32),
            pltpu.VMEM_SHARED((NS * TBL,), jnp.int32),
            pltpu.VMEM((NS * SLICE,), jnp.int32),
            pltpu.VMEM((NS * SLICE,), jnp.int32),
            pltpu.VMEM((SLICE,), jnp.int32),
            pltpu.VMEM((SLICE,), jnp.int32),
            pltpu.SemaphoreType.DMA,
        ],
    )
    def seg_kernel(x_hbm, m_hbm, sum_hbm, max_hbm,
                   xbuf, mbuf, acc_s, acc_m, mslab, fr_tbl, lp_tbl,
                   fr_sh, lp_sh, frst, lpst, fr96, lp96, sem):
        tid = lax.axis_index("s")
        cid = lax.axis_index("c")
        w = tid * NC + cid
        seg_lo = w * SEG_PER_W
        gbase = tid * cnt
        iota = lax.iota(jnp.int32, LANES)
        zero16 = jnp.zeros((LANES,), jnp.int32)

        @pl.loop(0, TBL // LANES)
        def _clr(i):
            fr_tbl[pl.ds(i * LANES, LANES)] = zero16
            lp_tbl[pl.ds(i * LANES, LANES)] = zero16

        @pl.when(tid == 0)
        def _():
            mslab[pl.ds(0, LANES)] = jnp.full((LANES,), -1, jnp.int32)
            pltpu.sync_copy(m_hbm.at[pl.ds(0, cnt + 8)],
                            mslab.at[pl.ds(8, cnt + 8)])

        @pl.when(tid == NS - 1)
        def _():
            pltpu.sync_copy(m_hbm.at[pl.ds(gbase - 8, cnt + 8)],
                            mslab.at[pl.ds(0, cnt + 8)])
            mslab[pl.ds(cnt + 8, LANES)] = jnp.full((LANES,), -2, jnp.int32)

        @pl.when(jnp.logical_and(tid != 0, tid != NS - 1))
        def _():
            pltpu.sync_copy(m_hbm.at[pl.ds(gbase - 8, cnt + 16)],
                            mslab.at[pl.ds(0, cnt + 16)])

        @pl.loop(0, cnt, step=LANES, unroll=4)
        def _scan(i):
            prev = mslab[pl.ds(i + 7, LANES)]
            curv = mslab[pl.ds(i + 8, LANES)]
            nxt = mslab[pl.ds(i + 9, LANES)]
            rowp1 = iota + (gbase + i + 1)
            plsc.store_scatter(fr_tbl, [curv], rowp1, mask=curv != prev)
            plsc.store_scatter(lp_tbl, [curv], rowp1, mask=curv != nxt)

        pltpu.sync_copy(fr_tbl, fr_sh.at[pl.ds(tid * TBL, TBL)])
        pltpu.sync_copy(lp_tbl, lp_sh.at[pl.ds(tid * TBL, TBL)])
        plsc.subcore_barrier()

        copies = []
        for t in range(NS):
            copies.append(pltpu.async_copy(
                fr_sh.at[pl.ds(t * TBL + seg_lo, SLICE)],
                frst.at[pl.ds(t * SLICE, SLICE)], sem))
            copies.append(pltpu.async_copy(
                lp_sh.at[pl.ds(t * TBL + seg_lo, SLICE)],
                lpst.at[pl.ds(t * SLICE, SLICE)], sem))
        for c in copies:
            c.wait()

        rmin = jnp.full((LANES,), BIG, jnp.int32)
        rmax = zero16
        for j in range(SLICE // LANES):
            sl = pl.ds(j * LANES, LANES)
            fv = frst[pl.ds(j * LANES, LANES)]
            lv = lpst[pl.ds(j * LANES, LANES)]
            for t in range(1, NS):
                fv = fv + frst[pl.ds(t * SLICE + j * LANES, LANES)]
                lv = lv + lpst[pl.ds(t * SLICE + j * LANES, LANES)]
            fr96[sl] = fv
            lp96[sl] = lv
            if j < SEG_PER_W // LANES:
                rmin = jnp.minimum(rmin, jnp.where(fv == 0, BIG, fv))
                rmax = jnp.maximum(rmax, lv)

        r0 = -plsc.cummax(-rmin)[LANES - 1] - 1
        r1 = plsc.cummax(rmax)[LANES - 1]

        zeros = jnp.zeros((LANES,), jnp.float32)
        ninf = jnp.full((LANES,), NEG_INF, jnp.float32)

        @pl.loop(0, SEG_PER_W)
        def _init(s):
            for j in range(NVEC):
                acc_s[s, pl.ds(j * LANES, LANES)] = zeros
                acc_m[s, pl.ds(j * LANES, LANES)] = ninf

        a0 = pl.multiple_of(jnp.maximum((r0 // 8) * 8, 0), 8)
        nchunks = jnp.maximum((r1 - a0 + CHUNK - 1) // CHUNK, 0)

        @pl.loop(0, nchunks, init_carry=jnp.int32(0))
        def _chunks(k, cur):
            s_un = a0 + k * CHUNK
            c0 = pl.multiple_of(jnp.minimum(s_un, N - CHUNK), 8)
            pltpu.sync_copy(x_hbm.at[pl.ds(c0, CHUNK)], xbuf)
            pltpu.sync_copy(m_hbm.at[pl.ds(c0, CHUNK)],
                            mbuf.at[pl.ds(0, CHUNK)])
            lo = jnp.maximum(r0, s_un) - c0
            hi = jnp.minimum(r1, s_un + CHUNK) - c0

            hi0 = jnp.maximum(hi - 1, 0)
            m_last = mbuf[pl.ds(hi0, LANES)][0] - seg_lo
            m_last = jnp.minimum(jnp.maximum(m_last, cur - 1), SEG_PER_W - 1)

            @pl.loop(cur, m_last + 1)
            def _runs(si):
                st = jnp.maximum(fr96[pl.ds(si, LANES)][0] - 1 - c0, lo)
                en = jnp.minimum(lp96[pl.ds(si, LANES)][0] - c0, hi)

                accs = tuple(acc_s[si, pl.ds(j * LANES, LANES)]
                             for j in range(NVEC))
                accm = tuple(acc_m[si, pl.ds(j * LANES, LANES)]
                             for j in range(NVEC))

                @plsc.parallel_loop(st, en, unroll=8, carry=accs + accm)
                def out(r, carry):
                    a = carry[:NVEC]
                    m = carry[NVEC:]
                    vs = tuple(xbuf[r, pl.ds(j * LANES, LANES)]
                               for j in range(NVEC))
                    a = tuple(a[j] + vs[j] for j in range(NVEC))
                    m = tuple(jnp.maximum(m[j], vs[j]) for j in range(NVEC))
                    return a + m

                for j in range(NVEC):
                    acc_s[si, pl.ds(j * LANES, LANES)] = out[j]
                    acc_m[si, pl.ds(j * LANES, LANES)] = out[NVEC + j]

            return jnp.maximum(m_last, cur)

        pltpu.sync_copy(acc_s, sum_hbm.at[pl.ds(seg_lo, SEG_PER_W)])
        pltpu.sync_copy(acc_m, max_hbm.at[pl.ds(seg_lo, SEG_PER_W)])

    return seg_kernel(x, m32)


def _tc_merge(seg_max, seg_sum, W_merge, b_merge):
    B, D = seg_max.shape

    def body(mx_ref, sm_ref, w_ref, b_ref, o_ref):
        mx = mx_ref[...]
        mx = jnp.where(jnp.isfinite(mx), mx, 0.0)
        acc = jnp.dot(mx, w_ref[0:D, :], preferred_element_type=jnp.float32)
        acc = acc + jnp.dot(sm_ref[...], w_ref[D:2 * D, :],
                            preferred_element_type=jnp.float32)
        o_ref[...] = acc + b_ref[...]

    return pl.pallas_call(
        body,
        out_shape=jax.ShapeDtypeStruct((B, W_merge.shape[1]), jnp.float32),
    )(seg_max, seg_sum, W_merge, b_merge)


def kernel(x, membership, W_merge, b_merge):
    m32 = membership.astype(jnp.int32)
    seg_sum, seg_max = _sc_graph_readout(x, m32)
    return _tc_merge(seg_max, seg_sum, W_merge, jnp.reshape(b_merge, (1, -1)))


# speedup vs baseline: 10.1531x; 1.5241x over previous
import functools

import jax
import jax.numpy as jnp
from jax import lax
from jax.experimental import pallas as pl
from jax.experimental.pallas import tpu as pltpu
from jax.experimental.pallas import tpu_sc as plsc

B_SEG = 2048
NC, NS = 2, 16
NW = NC * NS
SEG_PER_W = B_SEG // NW
CHUNK = 256
LANES = 16
NVEC = 8
TBL = 2304
SLICE = 96
NEG_INF = float("-inf")
BIG = jnp.int32(2 ** 30)


def _sc_graph_readout(x, m32):
    N, D = x.shape
    cnt = N // NS
    mesh = plsc.VectorSubcoreMesh(core_axis_name="c", subcore_axis_name="s")

    @functools.partial(
        pl.kernel,
        out_type=(
            jax.ShapeDtypeStruct((B_SEG, D), jnp.float32),
            jax.ShapeDtypeStruct((B_SEG, D), jnp.float32),
        ),
        mesh=mesh,
        compiler_params=pltpu.CompilerParams(needs_layout_passes=False),
        scratch_types=[
            pltpu.VMEM((CHUNK, D), jnp.float32),
            pltpu.VMEM((CHUNK, D), jnp.float32),
            pltpu.VMEM((CHUNK + LANES,), jnp.int32),
            pltpu.VMEM((CHUNK + LANES,), jnp.int32),
            pltpu.VMEM((SEG_PER_W, D), jnp.float32),
            pltpu.VMEM((SEG_PER_W, D), jnp.float32),
            pltpu.VMEM((cnt + 2 * LANES,), jnp.int32),
            pltpu.VMEM((TBL,), jnp.int32),
            pltpu.VMEM((TBL,), jnp.int32),
            pltpu.VMEM_SHARED((NS * TBL,), jnp.int32),
            pltpu.VMEM_SHARED((NS * TBL,), jnp.int32),
            pltpu.VMEM((NS * SLICE,), jnp.int32),
            pltpu.VMEM((NS * SLICE,), jnp.int32),
            pltpu.VMEM((SLICE,), jnp.int32),
            pltpu.VMEM((SLICE,), jnp.int32),
            pltpu.SemaphoreType.DMA,
            pltpu.SemaphoreType.DMA,
            pltpu.SemaphoreType.DMA,
        ],
    )
    def seg_kernel(x_hbm, m_hbm, sum_hbm, max_hbm,
                   xbuf0, xbuf1, mbuf0, mbuf1, acc_s, acc_m, mslab,
                   fr_tbl, lp_tbl, fr_sh, lp_sh, frst, lpst, fr96, lp96,
                   sem, semb0, semb1):
        tid = lax.axis_index("s")
        cid = lax.axis_index("c")
        w = tid * NC + cid
        seg_lo = w * SEG_PER_W
        gbase = tid * cnt
        iota = lax.iota(jnp.int32, LANES)
        zero16 = jnp.zeros((LANES,), jnp.int32)

        @pl.loop(0, TBL // LANES)
        def _clr(i):
            fr_tbl[pl.ds(i * LANES, LANES)] = zero16
            lp_tbl[pl.ds(i * LANES, LANES)] = zero16

        @pl.when(tid == 0)
        def _():
            mslab[pl.ds(0, LANES)] = jnp.full((LANES,), -1, jnp.int32)
            pltpu.sync_copy(m_hbm.at[pl.ds(0, cnt + 8)],
                            mslab.at[pl.ds(8, cnt + 8)])

        @pl.when(tid == NS - 1)
        def _():
            pltpu.sync_copy(m_hbm.at[pl.ds(gbase - 8, cnt + 8)],
                            mslab.at[pl.ds(0, cnt + 8)])
            mslab[pl.ds(cnt + 8, LANES)] = jnp.full((LANES,), -2, jnp.int32)

        @pl.when(jnp.logical_and(tid != 0, tid != NS - 1))
        def _():
            pltpu.sync_copy(m_hbm.at[pl.ds(gbase - 8, cnt + 16)],
                            mslab.at[pl.ds(0, cnt + 16)])

        @pl.loop(0, cnt, step=LANES, unroll=4)
        def _scan(i):
            prev = mslab[pl.ds(i + 7, LANES)]
            curv = mslab[pl.ds(i + 8, LANES)]
            nxt = mslab[pl.ds(i + 9, LANES)]
            rowp1 = iota + (gbase + i + 1)
            plsc.store_scatter(fr_tbl, [curv], rowp1, mask=curv != prev)
            plsc.store_scatter(lp_tbl, [curv], rowp1, mask=curv != nxt)

        pltpu.sync_copy(fr_tbl, fr_sh.at[pl.ds(tid * TBL, TBL)])
        pltpu.sync_copy(lp_tbl, lp_sh.at[pl.ds(tid * TBL, TBL)])
        plsc.subcore_barrier()

        copies = []
        for t in range(NS):
            copies.append(pltpu.async_copy(
                fr_sh.at[pl.ds(t * TBL + seg_lo, SLICE)],
                frst.at[pl.ds(t * SLICE, SLICE)], sem))
            copies.append(pltpu.async_copy(
                lp_sh.at[pl.ds(t * TBL + seg_lo, SLICE)],
                lpst.at[pl.ds(t * SLICE, SLICE)], sem))
        for c in copies:
            c.wait()

        rmin = jnp.full((LANES,), BIG, jnp.int32)
        rmax = zero16
        for j in range(SLICE // LANES):
            sl = pl.ds(j * LANES, LANES)
            fv = frst[pl.ds(j * LANES, LANES)]
            lv = lpst[pl.ds(j * LANES, LANES)]
            for t in range(1, NS):
                fv = fv + frst[pl.ds(t * SLICE + j * LANES, LANES)]
                lv = lv + lpst[pl.ds(t * SLICE + j * LANES, LANES)]
            fr96[sl] = fv
            lp96[sl] = lv
            if j < SEG_PER_W // LANES:
                rmin = jnp.minimum(rmin, jnp.where(fv == 0, BIG, fv))
                rmax = jnp.maximum(rmax, lv)

        r0 = -plsc.cummax(-rmin)[LANES - 1] - 1
        r1 = plsc.cummax(rmax)[LANES - 1]

        zeros = jnp.zeros((LANES,), jnp.float32)
        ninf = jnp.full((LANES,), NEG_INF, jnp.float32)

        @pl.loop(0, SEG_PER_W)
        def _init(s):
            for j in range(NVEC):
                acc_s[s, pl.ds(j * LANES, LANES)] = zeros
                acc_m[s, pl.ds(j * LANES, LANES)] = ninf

        a0 = pl.multiple_of(jnp.maximum((r0 // 8) * 8, 0), 8)
        nchunks = jnp.maximum((r1 - a0 + CHUNK - 1) // CHUNK, 0)

        def coff(k):
            return pl.multiple_of(jnp.minimum(a0 + k * CHUNK, N - CHUNK), 8)

        def start_copy(k, xb, mb, sb):
            c0 = coff(k)
            pltpu.async_copy(x_hbm.at[pl.ds(c0, CHUNK)], xb, sb)
            pltpu.async_copy(m_hbm.at[pl.ds(c0, CHUNK)],
                             mb.at[pl.ds(0, CHUNK)], sb)

        def wait_copy(k, xb, mb, sb):
            c0 = coff(k)
            pltpu.make_async_copy(x_hbm.at[pl.ds(c0, CHUNK)], xb, sb).wait()
            pltpu.make_async_copy(m_hbm.at[pl.ds(c0, CHUNK)],
                                  mb.at[pl.ds(0, CHUNK)], sb).wait()

        bufs = ((xbuf0, mbuf0, semb0), (xbuf1, mbuf1, semb1))
        for b in range(2):
            xb, mb, sb = bufs[b]

            @pl.when(jnp.int32(b) < nchunks)
            def _():
                start_copy(jnp.int32(b), xb, mb, sb)

        nsuper = (nchunks + 1) // 2

        @pl.loop(0, nsuper, init_carry=jnp.int32(0))
        def _chunks(ks, cur):
            for b in range(2):
                xb, mb, sb = bufs[b]
                k = ks * 2 + b
                valid = k < nchunks

                @pl.when(valid)
                def _():
                    wait_copy(k, xb, mb, sb)

                c0 = coff(k)
                s_un = a0 + k * CHUNK
                lo = jnp.maximum(r0, s_un) - c0
                hi = jnp.minimum(r1, s_un + CHUNK) - c0
                hi = jnp.where(valid, hi, lo)

                hi0 = jnp.maximum(hi - 1, 0)
                m_last = mb[pl.ds(hi0, LANES)][0] - seg_lo
                m_last = jnp.minimum(jnp.maximum(m_last, cur - 1),
                                     SEG_PER_W - 1)
                m_last = jnp.where(valid, m_last, cur - 1)

                @pl.loop(cur, m_last + 1)
                def _runs(si):
                    st = jnp.maximum(fr96[pl.ds(si, LANES)][0] - 1 - c0, lo)
                    en = jnp.minimum(lp96[pl.ds(si, LANES)][0] - c0, hi)

                    accs = tuple(acc_s[si, pl.ds(j * LANES, LANES)]
                                 for j in range(NVEC))
                    accm = tuple(acc_m[si, pl.ds(j * LANES, LANES)]
                                 for j in range(NVEC))

                    @plsc.parallel_loop(st, en, unroll=8, carry=accs + accm)
                    def out(r, carry):
                        a = carry[:NVEC]
                        m = carry[NVEC:]
                        vs = tuple(xb[r, pl.ds(j * LANES, LANES)]
                                   for j in range(NVEC))
                        a = tuple(a[j] + vs[j] for j in range(NVEC))
                        m = tuple(jnp.maximum(m[j], vs[j])
                                  for j in range(NVEC))
                        return a + m

                    for j in range(NVEC):
                        acc_s[si, pl.ds(j * LANES, LANES)] = out[j]
                        acc_m[si, pl.ds(j * LANES, LANES)] = out[NVEC + j]

                @pl.when(k + 2 < nchunks)
                def _():
                    start_copy(k + 2, xb, mb, sb)

                cur = jnp.maximum(m_last, cur)
            return cur

        pltpu.sync_copy(acc_s, sum_hbm.at[pl.ds(seg_lo, SEG_PER_W)])
        pltpu.sync_copy(acc_m, max_hbm.at[pl.ds(seg_lo, SEG_PER_W)])

    return seg_kernel(x, m32)


def _tc_merge(seg_max, seg_sum, W_merge, b_merge):
    B, D = seg_max.shape

    def body(mx_ref, sm_ref, w_ref, b_ref, o_ref):
        mx = mx_ref[...]
        mx = jnp.where(jnp.isfinite(mx), mx, 0.0)
        acc = jnp.dot(mx, w_ref[0:D, :], preferred_element_type=jnp.float32)
        acc = acc + jnp.dot(sm_ref[...], w_ref[D:2 * D, :],
                            preferred_element_type=jnp.float32)
        o_ref[...] = acc + b_ref[...]

    return pl.pallas_call(
        body,
        out_shape=jax.ShapeDtypeStruct((B, W_merge.shape[1]), jnp.float32),
    )(seg_max, seg_sum, W_merge, b_merge)


def kernel(x, membership, W_merge, b_merge):
    m32 = membership.astype(jnp.int32)
    seg_sum, seg_max = _sc_graph_readout(x, m32)
    return _tc_merge(seg_max, seg_sum, W_merge, jnp.reshape(b_merge, (1, -1)))


# speedup vs baseline: 10.1827x; 1.0029x over previous
import functools

import jax
import jax.numpy as jnp
from jax import lax
from jax.experimental import pallas as pl
from jax.experimental.pallas import tpu as pltpu
from jax.experimental.pallas import tpu_sc as plsc

B_SEG = 2048
NC, NS = 2, 16
NW = NC * NS
SEG_PER_W = B_SEG // NW
CHUNK = 256
LANES = 16
NVEC = 8
TBL = 2304
SLICE = 96
NEG_INF = float("-inf")
BIG = jnp.int32(2 ** 30)


def _sc_graph_readout(x, m32):
    N, D = x.shape
    cnt = N // NS
    mesh = plsc.VectorSubcoreMesh(core_axis_name="c", subcore_axis_name="s")

    @functools.partial(
        pl.kernel,
        out_type=(
            jax.ShapeDtypeStruct((B_SEG, D), jnp.float32),
            jax.ShapeDtypeStruct((B_SEG, D), jnp.float32),
        ),
        mesh=mesh,
        compiler_params=pltpu.CompilerParams(needs_layout_passes=False),
        scratch_types=[
            pltpu.VMEM((CHUNK, D), jnp.float32),
            pltpu.VMEM((CHUNK, D), jnp.float32),
            pltpu.VMEM((CHUNK + LANES,), jnp.int32),
            pltpu.VMEM((CHUNK + LANES,), jnp.int32),
            pltpu.VMEM((SEG_PER_W, D), jnp.float32),
            pltpu.VMEM((SEG_PER_W, D), jnp.float32),
            pltpu.VMEM((cnt + 2 * LANES,), jnp.int32),
            pltpu.VMEM((TBL,), jnp.int32),
            pltpu.VMEM((TBL,), jnp.int32),
            pltpu.VMEM_SHARED((NS * TBL,), jnp.int32),
            pltpu.VMEM_SHARED((NS * TBL,), jnp.int32),
            pltpu.VMEM((NS * SLICE,), jnp.int32),
            pltpu.VMEM((NS * SLICE,), jnp.int32),
            pltpu.VMEM((SLICE,), jnp.int32),
            pltpu.VMEM((SLICE,), jnp.int32),
            pltpu.SemaphoreType.DMA,
            pltpu.SemaphoreType.DMA,
            pltpu.SemaphoreType.DMA,
        ],
    )
    def seg_kernel(x_hbm, m_hbm, sum_hbm, max_hbm,
                   xbuf0, xbuf1, mbuf0, mbuf1, acc_s, acc_m, mslab,
                   fr_tbl, lp_tbl, fr_sh, lp_sh, frst, lpst, fr96, lp96,
                   sem, semb0, semb1):
        tid = lax.axis_index("s")
        cid = lax.axis_index("c")
        w = tid * NC + cid
        seg_lo = w * SEG_PER_W
        gbase = tid * cnt
        iota = lax.iota(jnp.int32, LANES)
        zero16 = jnp.zeros((LANES,), jnp.int32)

        @pl.loop(0, TBL // LANES)
        def _clr(i):
            fr_tbl[pl.ds(i * LANES, LANES)] = zero16
            lp_tbl[pl.ds(i * LANES, LANES)] = zero16

        @pl.when(tid == 0)
        def _():
            mslab[pl.ds(0, LANES)] = jnp.full((LANES,), -1, jnp.int32)
            pltpu.sync_copy(m_hbm.at[pl.ds(0, cnt + 8)],
                            mslab.at[pl.ds(8, cnt + 8)])

        @pl.when(tid == NS - 1)
        def _():
            pltpu.sync_copy(m_hbm.at[pl.ds(gbase - 8, cnt + 8)],
                            mslab.at[pl.ds(0, cnt + 8)])
            mslab[pl.ds(cnt + 8, LANES)] = jnp.full((LANES,), -2, jnp.int32)

        @pl.when(jnp.logical_and(tid != 0, tid != NS - 1))
        def _():
            pltpu.sync_copy(m_hbm.at[pl.ds(gbase - 8, cnt + 16)],
                            mslab.at[pl.ds(0, cnt + 16)])

        @pl.loop(0, cnt, step=LANES, unroll=8)
        def _scan(i):
            prev = mslab[pl.ds(i + 7, LANES)]
            curv = mslab[pl.ds(i + 8, LANES)]
            nxt = mslab[pl.ds(i + 9, LANES)]
            rowp1 = iota + (gbase + i + 1)
            plsc.store_scatter(fr_tbl, [curv], rowp1, mask=curv != prev)
            plsc.store_scatter(lp_tbl, [curv], rowp1, mask=curv != nxt)

        pltpu.sync_copy(fr_tbl, fr_sh.at[pl.ds(tid * TBL, TBL)])
        pltpu.sync_copy(lp_tbl, lp_sh.at[pl.ds(tid * TBL, TBL)])
        plsc.subcore_barrier()

        copies = []
        for t in range(NS):
            copies.append(pltpu.async_copy(
                fr_sh.at[pl.ds(t * TBL + seg_lo, SLICE)],
                frst.at[pl.ds(t * SLICE, SLICE)], sem))
            copies.append(pltpu.async_copy(
                lp_sh.at[pl.ds(t * TBL + seg_lo, SLICE)],
                lpst.at[pl.ds(t * SLICE, SLICE)], sem))
        for c in copies:
            c.wait()

        rmin = jnp.full((LANES,), BIG, jnp.int32)
        rmax = zero16
        for j in range(SLICE // LANES):
            sl = pl.ds(j * LANES, LANES)
            fv = frst[pl.ds(j * LANES, LANES)]
            lv = lpst[pl.ds(j * LANES, LANES)]
            for t in range(1, NS):
                fv = fv + frst[pl.ds(t * SLICE + j * LANES, LANES)]
                lv = lv + lpst[pl.ds(t * SLICE + j * LANES, LANES)]
            fr96[sl] = fv
            lp96[sl] = lv
            if j < SEG_PER_W // LANES:
                rmin = jnp.minimum(rmin, jnp.where(fv == 0, BIG, fv))
                rmax = jnp.maximum(rmax, lv)

        r0 = -plsc.cummax(-rmin)[LANES - 1] - 1
        r1 = plsc.cummax(rmax)[LANES - 1]

        zeros = jnp.zeros((LANES,), jnp.float32)
        ninf = jnp.full((LANES,), NEG_INF, jnp.float32)

        @pl.loop(0, SEG_PER_W)
        def _init(s):
            for j in range(NVEC):
                acc_s[s, pl.ds(j * LANES, LANES)] = zeros
                acc_m[s, pl.ds(j * LANES, LANES)] = ninf

        a0 = pl.multiple_of(jnp.maximum((r0 // 8) * 8, 0), 8)
        nchunks = jnp.maximum((r1 - a0 + CHUNK - 1) // CHUNK, 0)

        def coff(k):
            return pl.multiple_of(jnp.minimum(a0 + k * CHUNK, N - CHUNK), 8)

        def start_copy(k, xb, mb, sb):
            c0 = coff(k)
            pltpu.async_copy(x_hbm.at[pl.ds(c0, CHUNK)], xb, sb)
            pltpu.async_copy(m_hbm.at[pl.ds(c0, CHUNK)],
                             mb.at[pl.ds(0, CHUNK)], sb)

        def wait_copy(k, xb, mb, sb):
            c0 = coff(k)
            pltpu.make_async_copy(x_hbm.at[pl.ds(c0, CHUNK)], xb, sb).wait()
            pltpu.make_async_copy(m_hbm.at[pl.ds(c0, CHUNK)],
                                  mb.at[pl.ds(0, CHUNK)], sb).wait()

        bufs = ((xbuf0, mbuf0, semb0), (xbuf1, mbuf1, semb1))
        for b in range(2):
            xb, mb, sb = bufs[b]

            @pl.when(jnp.int32(b) < nchunks)
            def _():
                start_copy(jnp.int32(b), xb, mb, sb)

        nsuper = (nchunks + 1) // 2

        @pl.loop(0, nsuper, init_carry=jnp.int32(0))
        def _chunks(ks, cur):
            for b in range(2):
                xb, mb, sb = bufs[b]
                k = ks * 2 + b
                valid = k < nchunks

                @pl.when(valid)
                def _():
                    wait_copy(k, xb, mb, sb)

                c0 = coff(k)
                s_un = a0 + k * CHUNK
                lo = jnp.maximum(r0, s_un) - c0
                hi = jnp.minimum(r1, s_un + CHUNK) - c0
                hi = jnp.where(valid, hi, lo)

                hi0 = jnp.maximum(hi - 1, 0)
                m_last = mb[pl.ds(hi0, LANES)][0] - seg_lo
                m_last = jnp.minimum(jnp.maximum(m_last, cur - 1),
                                     SEG_PER_W - 1)
                m_last = jnp.where(valid, m_last, cur - 1)

                @pl.loop(cur, m_last + 1)
                def _runs(si):
                    st = jnp.maximum(fr96[pl.ds(si, LANES)][0] - 1 - c0, lo)
                    en = jnp.minimum(lp96[pl.ds(si, LANES)][0] - c0, hi)

                    accs = tuple(acc_s[si, pl.ds(j * LANES, LANES)]
                                 for j in range(NVEC))
                    accm = tuple(acc_m[si, pl.ds(j * LANES, LANES)]
                                 for j in range(NVEC))

                    @plsc.parallel_loop(st, en, unroll=8, carry=accs + accm)
                    def out(r, carry):
                        a = carry[:NVEC]
                        m = carry[NVEC:]
                        vs = tuple(xb[r, pl.ds(j * LANES, LANES)]
                                   for j in range(NVEC))
                        a = tuple(a[j] + vs[j] for j in range(NVEC))
                        m = tuple(jnp.maximum(m[j], vs[j])
                                  for j in range(NVEC))
                        return a + m

                    for j in range(NVEC):
                        acc_s[si, pl.ds(j * LANES, LANES)] = out[j]
                        acc_m[si, pl.ds(j * LANES, LANES)] = out[NVEC + j]

                @pl.when(k + 2 < nchunks)
                def _():
                    start_copy(k + 2, xb, mb, sb)

                cur = jnp.maximum(m_last, cur)
            return cur

        pltpu.sync_copy(acc_s, sum_hbm.at[pl.ds(seg_lo, SEG_PER_W)])
        pltpu.sync_copy(acc_m, max_hbm.at[pl.ds(seg_lo, SEG_PER_W)])

    return seg_kernel(x, m32)


def _tc_merge(seg_max, seg_sum, W_merge, b_merge):
    B, D = seg_max.shape

    def body(mx_ref, sm_ref, w_ref, b_ref, o_ref):
        mx = mx_ref[...]
        mx = jnp.where(jnp.isfinite(mx), mx, 0.0)
        acc = jnp.dot(mx, w_ref[0:D, :], preferred_element_type=jnp.float32)
        acc = acc + jnp.dot(sm_ref[...], w_ref[D:2 * D, :],
                            preferred_element_type=jnp.float32)
        o_ref[...] = acc + b_ref[...]

    return pl.pallas_call(
        body,
        out_shape=jax.ShapeDtypeStruct((B, W_merge.shape[1]), jnp.float32),
    )(seg_max, seg_sum, W_merge, b_merge)


def kernel(x, membership, W_merge, b_merge):
    m32 = membership.astype(jnp.int32)
    seg_sum, seg_max = _sc_graph_readout(x, m32)
    return _tc_merge(seg_max, seg_sum, W_merge, jnp.reshape(b_merge, (1, -1)))
